# baseline (speedup 1.0000x reference)
import functools

import jax
import jax.numpy as jnp
from jax import lax
from jax.experimental import pallas as pl
from jax.experimental.pallas import tpu as pltpu
from jax.experimental.pallas import tpu_sc as plsc

NUM_CORES = 2
NUM_SUBCORES = 16
NUM_TILES = NUM_CORES * NUM_SUBCORES
GROUP = 8


@functools.lru_cache(maxsize=None)
def _build(Bt, S, D):
    rows_per_tile = Bt // NUM_TILES
    n_groups = rows_per_tile // GROUP
    assert n_groups % 2 == 0
    mesh = plsc.VectorSubcoreMesh(core_axis_name="c", subcore_axis_name="s")

    def body(tok_ref, table_ref, out_ref, idx_v, rows0, rows1, g0, g1, o0, o1):
        wid = lax.axis_index("s") * NUM_CORES + lax.axis_index("c")
        row0 = wid * rows_per_tile
        pltpu.sync_copy(tok_ref.at[pl.ds(row0, rows_per_tile)], idx_v)
        rows = (rows0, rows1)
        gsem = (g0, g1)
        osem = (o0, o1)

        def fire_gathers(g, buf):
            for k in range(GROUP):
                pltpu.async_copy(
                    table_ref.at[idx_v.at[g * GROUP + k]],
                    rows[buf].at[k], gsem[buf])

        def wait_gathers(buf):
            for k in range(GROUP):
                pltpu.make_async_copy(
                    table_ref.at[idx_v.at[k]],
                    rows[buf].at[k], gsem[buf]).wait()

        def fire_out(g, buf):
            pltpu.async_copy(
                rows[buf], out_ref.at[pl.ds(row0 + g * GROUP, GROUP)],
                osem[buf])

        def wait_out(g, buf):
            pltpu.make_async_copy(
                rows[buf], out_ref.at[pl.ds(row0 + g * GROUP, GROUP)],
                osem[buf]).wait()

        fire_gathers(0, 0)
        fire_gathers(1, 1)
        wait_gathers(0)
        fire_out(0, 0)

        @pl.loop(1, n_groups - 1, step=2)
        def _pair(g):
            wait_out(g - 1, 0)
            fire_gathers(g + 1, 0)
            wait_gathers(1)
            fire_out(g, 1)

            wait_out(g, 1)
            fire_gathers(g + 2, 1)
            wait_gathers(0)
            fire_out(g + 1, 0)

        g_last = n_groups - 1
        wait_gathers(1)
        fire_out(g_last, 1)
        wait_out(g_last - 1, 0)
        wait_out(g_last, 1)

    return pl.kernel(
        body,
        out_type=jax.ShapeDtypeStruct((Bt, S, D), jnp.float32),
        mesh=mesh,
        scratch_types=[
            pltpu.VMEM((Bt // NUM_TILES, S), jnp.int32),
            pltpu.VMEM((GROUP, S, D), jnp.float32),
            pltpu.VMEM((GROUP, S, D), jnp.float32),
            pltpu.SemaphoreType.DMA,
            pltpu.SemaphoreType.DMA,
            pltpu.SemaphoreType.DMA,
            pltpu.SemaphoreType.DMA,
        ],
        compiler_params=pltpu.CompilerParams(use_tc_tiling_on_sc=False),
    )


def kernel(token_ids, weight):
    Bt, S = token_ids.shape
    V, D = weight.shape
    tok = token_ids.astype(jnp.int32)
    return _build(Bt, S, D)(tok, weight)
